# baseline (speedup 1.0000x reference)
import functools

import jax
import jax.numpy as jnp
from jax import lax
from jax.experimental import pallas as pl
from jax.experimental.pallas import tpu as pltpu
from jax.experimental.pallas import tpu_sc as plsc

_B = 16384
_F = 26
_V = 100000
_D = 16
_C = 13
_OUT = _C + _F * _D
_N = 128


def _make_sc_kernel():
    info = plsc.get_sparse_core_info()
    nw = info.num_cores * info.num_subcores
    nb = _B // nw
    n_sub = nb // _N

    mesh = plsc.VectorSubcoreMesh(core_axis_name="c", subcore_axis_name="s")

    @functools.partial(
        pl.kernel,
        mesh=mesh,
        out_type=jax.ShapeDtypeStruct((_B * _OUT,), jnp.float32),
        scratch_types=[
            pltpu.VMEM((_N * _F,), jnp.int32),
            pltpu.VMEM((_F, _N), jnp.int32),
            pltpu.VMEM((_F * _N, _D), jnp.float32),
            pltpu.VMEM((_N, 16), jnp.float32),
            pltpu.VMEM((_N * _OUT,), jnp.float32),
            pltpu.SemaphoreType.DMA,
        ],
        compiler_params=pltpu.CompilerParams(
            use_tc_tiling_on_sc=False, needs_layout_passes=False
        ),
    )
    def k(cont_hbm, cat_hbm, w_hbm, out_hbm, cat_v, idx_v, gat_v, cont_v,
          row_v, sem):
        wid = lax.axis_index("s") * info.num_cores + lax.axis_index("c")
        lane = lax.iota(jnp.int32, 16)
        stride_pat = lane * _F

        def sub_chunk(s, _):
            rbase = wid * nb + s * _N
            pltpu.sync_copy(cat_hbm.at[pl.ds(rbase * _F, _N * _F)], cat_v)
            ccopy = pltpu.async_copy(
                cont_hbm.at[pl.ds(rbase, _N), :], cont_v, sem
            )
            for f in range(_F):
                for g in range(_N // 16):
                    src = stride_pat + (g * 16 * _F + f)
                    vals = plsc.load_gather(cat_v, [src])
                    idx_v[f, pl.ds(g * 16, 16)] = vals + f * _V
            copies = [
                pltpu.async_copy(
                    w_hbm.at[idx_v.at[f]],
                    gat_v.at[pl.ds(f * _N, _N), :],
                    sem,
                )
                for f in range(_F)
            ]
            for c in copies:
                c.wait()
            ccopy.wait()

            def fill_row(r, _):
                o = r * _OUT
                row_v[pl.ds(o, 16)] = cont_v[r, :]
                for f in range(_F):
                    row_v[pl.ds(o + _C + f * _D, _D)] = gat_v[f * _N + r, :]
                return ()

            lax.fori_loop(0, _N, fill_row, ())
            pltpu.sync_copy(row_v, out_hbm.at[pl.ds(rbase * _OUT, _N * _OUT)])
            return ()

        lax.fori_loop(0, n_sub, sub_chunk, ())

    return k


_sc_kernel = _make_sc_kernel()


def kernel(x_cont, x_cat, W):
    w_flat = W.reshape(_F * _V, _D)
    cat_flat = x_cat.reshape(-1)
    cont16 = jnp.pad(x_cont, ((0, 0), (0, 16 - _C)))
    out_flat = _sc_kernel(cont16, cat_flat, w_flat)
    return out_flat.reshape(_B, _OUT)
